# speedup vs baseline: 1.1165x; 1.1165x over previous
import functools

import jax
import jax.numpy as jnp
from jax.experimental import pallas as pl
from jax.experimental.pallas import tpu as pltpu

_HLL_M = 256.0
_ALPHA = 0.7213 / (1.0 + 1.079 / _HLL_M)


def _hll_count_f(regs_f):
    s = jnp.sum(jnp.exp2(-regs_f), axis=-1, keepdims=True)
    est = _ALPHA * _HLL_M * _HLL_M / s
    zeros = jnp.sum((regs_f == 0.0).astype(jnp.float32), axis=-1, keepdims=True)
    lin = _HLL_M * jnp.log(_HLL_M / jnp.maximum(zeros, 1.0))
    return jnp.where((est <= 2.5 * _HLL_M) & (zeros > 0), lin, est)


def _edge_body(ms1, ms2, md1, md2, hs1, hs2, hd1, hd2, cs, cd, wl, bl,
               sf_ref, stat_ref):
    def jac(a, b):
        return jnp.mean((a[...] == b[...]).astype(jnp.float32), axis=-1,
                        keepdims=True)

    def un(a, b):
        return _hll_count_f(jnp.maximum(a[...], b[...]).astype(jnp.float32))

    f0 = jac(ms1, md1) * un(hs1, hd1)
    i21 = jac(ms2, md1) * un(hs2, hd1)
    i12 = jac(ms1, md2) * un(hs1, hd2)
    i22 = jac(ms2, md2) * un(hs2, hd2)
    f1 = i21 - f0
    f2 = i12 - f0
    f3 = i22 - f0 - f1 - f2
    c1 = cs[...]
    c2 = cd[...]
    f4 = c2[:, 0:1] - f0 - f1
    f5 = c1[:, 0:1] - f0 - f2
    f6 = c2[:, 1:2] - f1 - f3 - f4
    f7 = c1[:, 1:2] - f2 - f3 - f5
    feats = jnp.concatenate([f0, f1, f2, f3, f4, f5, f6, f7], axis=1)
    feats = jnp.maximum(feats, 0.0)
    sf = jnp.dot(feats, wl[...], preferred_element_type=jnp.float32) + bl[...]
    sf_ref[...] = sf

    @pl.when(pl.program_id(0) == 0)
    def _():
        stat_ref[...] = jnp.zeros_like(stat_ref)

    stat_ref[...] += jnp.concatenate(
        [jnp.sum(sf, axis=0, keepdims=True),
         jnp.sum(sf * sf, axis=0, keepdims=True)], axis=0)


def _edge_stage(ms1, ms2, md1, md2, hs1, hs2, hd1, hd2, cs, cd, Wl, bl):
    e = ms1.shape[0]
    blk = 800 if e % 800 == 0 else e
    grid = e // blk
    mh_spec = pl.BlockSpec((blk, 128), lambda i: (i, 0))
    hl_spec = pl.BlockSpec((blk, 256), lambda i: (i, 0))
    c_spec = pl.BlockSpec((blk, 2), lambda i: (i, 0))
    w_spec = pl.BlockSpec((8, 8), lambda i: (0, 0))
    b_spec = pl.BlockSpec((1, 8), lambda i: (0, 0))
    sf, stat = pl.pallas_call(
        _edge_body,
        grid=(grid,),
        in_specs=[mh_spec] * 4 + [hl_spec] * 4 + [c_spec] * 2 + [w_spec, b_spec],
        out_specs=[pl.BlockSpec((blk, 8), lambda i: (i, 0)),
                   pl.BlockSpec((2, 8), lambda i: (0, 0))],
        out_shape=[jax.ShapeDtypeStruct((e, 8), jnp.float32),
                   jax.ShapeDtypeStruct((2, 8), jnp.float32)],
    )(ms1, ms2, md1, md2, hs1, hs2, hd1, hd2, cs, cd, Wl,
      bl.reshape(1, 8))
    return sf, stat


def kernel(x, edge_index, embed, W1, b1, W2, b2, Wl, bl, gamma, beta, Wf, bf,
           init_minhash, init_hll):
    n = x.shape[0]
    e = edge_index.shape[1]
    src = edge_index[0]
    dst = edge_index[1]

    mh1 = jnp.minimum(init_minhash,
                      jax.ops.segment_min(init_minhash[src], dst, num_segments=n))
    mh2 = jnp.minimum(mh1, jax.ops.segment_min(mh1[src], dst, num_segments=n))
    hl1 = jnp.maximum(init_hll,
                      jax.ops.segment_max(init_hll[src], dst, num_segments=n))
    hl2 = jnp.maximum(hl1, jax.ops.segment_max(hl1[src], dst, num_segments=n))

    cards = jnp.concatenate(
        [_hll_count_f(hl1.astype(jnp.float32)),
         _hll_count_f(hl2.astype(jnp.float32))], axis=1)

    deg = 1.0 + jax.ops.segment_sum(jnp.ones((e,), jnp.float32), dst,
                                    num_segments=n)
    dinv = 1.0 / jnp.sqrt(jnp.maximum(deg, 1.0))
    enorm = (dinv[src] * dinv[dst])[:, None]

    h = embed @ W1
    h = jax.nn.relu(jax.ops.segment_sum(h[src] * enorm, dst, num_segments=n)
                    + h * (dinv * dinv)[:, None] + b1)
    h = h @ W2
    h = jax.nn.relu(jax.ops.segment_sum(h[src] * enorm, dst, num_segments=n)
                    + h * (dinv * dinv)[:, None] + b2)

    sf, stat = _edge_stage(mh1[src], mh2[src], mh1[dst], mh2[dst],
                           hl1[src], hl2[src], hl1[dst], hl2[dst],
                           cards[src], cards[dst], Wl, bl)
    mean = stat[0] / e
    var = stat[1] / e - mean * mean
    sf = gamma * (sf - mean) / jnp.sqrt(var + 1e-5) + beta
    sf = jax.nn.relu(sf)
    sf_nodes = jax.ops.segment_sum(sf, dst, num_segments=n)

    out = jnp.concatenate([h, sf_nodes], axis=1) @ Wf + bf
    return jax.nn.sigmoid(out)


# speedup vs baseline: 1.1299x; 1.0120x over previous
import functools

import jax
import jax.numpy as jnp
from jax import lax
from jax.experimental import pallas as pl
from jax.experimental.pallas import tpu as pltpu
from jax.experimental.pallas import tpu_sc as plsc

_HLL_M = 256.0
_ALPHA = 0.7213 / (1.0 + 1.079 / _HLL_M)

_NC = 2
_NS = 16
_NW = _NC * _NS
_CAP = 5888
_EB = 32
_ECHUNK = 20000


def _make_partition(e, npw):
    mesh = plsc.VectorSubcoreMesh(core_axis_name="c", subcore_axis_name="s", num_cores=_NC, num_subcores=_NS)
    nchunk = (e + _ECHUNK - 1) // _ECHUNK
    assert e % 16 == 0

    @functools.partial(
        pl.kernel,
        out_type=[
            jax.ShapeDtypeStruct((_NW, _CAP), jnp.int32),
            jax.ShapeDtypeStruct((_NW, _CAP), jnp.int32),
            jax.ShapeDtypeStruct((_NW, 16), jnp.int32),
        ],
        mesh=mesh,
        scratch_types=[
            pltpu.VMEM((_ECHUNK,), jnp.int32),
            pltpu.VMEM((_ECHUNK,), jnp.int32),
            pltpu.VMEM((_CAP + 16,), jnp.int32),
            pltpu.VMEM((_CAP + 16,), jnp.int32),
            pltpu.VMEM((16,), jnp.int32),
        ],
        compiler_params=pltpu.CompilerParams(needs_layout_passes=False),
    )
    def part(src_hbm, dst_hbm, bsrc_hbm, bdstl_hbm, bcnt_hbm,
             src_c, dst_c, bsrc_v, bdstl_v, cnt_v):
        wid = lax.axis_index("s") * _NC + lax.axis_index("c")
        lo = wid * npw

        def memset(i, _):
            bsrc_v[pl.ds(i * 16, 16)] = jnp.zeros((16,), jnp.int32)
            bdstl_v[pl.ds(i * 16, 16)] = jnp.full((16,), npw, jnp.int32)
            return 0

        lax.fori_loop(0, (_CAP + 16) // 16, memset, 0)

        ptr = jnp.int32(0)
        for c in range(nchunk):
            csz = min(_ECHUNK, e - c * _ECHUNK)
            pltpu.sync_copy(src_hbm.at[pl.ds(c * _ECHUNK, csz)],
                            src_c.at[pl.ds(0, csz)])
            pltpu.sync_copy(dst_hbm.at[pl.ds(c * _ECHUNK, csz)],
                            dst_c.at[pl.ds(0, csz)])

            def scan(i, ptr):
                vd = dst_c[pl.ds(i * 16, 16)]
                vs = src_c[pl.ds(i * 16, 16)]
                m = (vd >= lo) & (vd < lo + npw)
                cs = plsc.cumsum(m.astype(jnp.int32))
                nmatch = cs[15]

                @pl.when(nmatch > 0)
                def _():
                    pos = cs - 1 + ptr
                    plsc.store_scatter(bdstl_v, [pos], vd - lo, mask=m)
                    plsc.store_scatter(bsrc_v, [pos], vs, mask=m)

                return ptr + nmatch

            ptr = lax.fori_loop(0, csz // 16, scan, ptr)

        iota = lax.iota(jnp.int32, 16)

        def selfloop(q, ptr):
            dl = q * 16 + iota
            m = dl < npw
            cs = plsc.cumsum(m.astype(jnp.int32))
            plsc.store_scatter(bdstl_v, [ptr + iota], dl, mask=m)
            plsc.store_scatter(bsrc_v, [ptr + iota], dl + lo, mask=m)
            return ptr + cs[15]

        ptr = lax.fori_loop(0, (npw + 15) // 16, selfloop, ptr)

        cnt_v[...] = jnp.full((16,), 1, jnp.int32) * ptr
        pltpu.sync_copy(cnt_v, bcnt_hbm.at[wid])
        pltpu.sync_copy(bsrc_v.at[pl.ds(0, _CAP)], bsrc_hbm.at[wid])
        pltpu.sync_copy(bdstl_v.at[pl.ds(0, _CAP)], bdstl_hbm.at[wid])

    return part


def _make_segreduce(npad, c, npw, op, dtype):
    mesh = plsc.VectorSubcoreMesh(core_axis_name="c", subcore_axis_name="s", num_cores=_NC, num_subcores=_NS)
    nj = c // 16
    eb = 16
    if op == "min":
        combine = jnp.minimum
        identity = 2147483647
    elif op == "max":
        combine = jnp.maximum
        identity = -2147483648
    else:
        combine = lambda a, b: a + b
        identity = 0.0

    @functools.partial(
        pl.kernel,
        out_type=jax.ShapeDtypeStruct((npad * c,), dtype),
        mesh=mesh,
        scratch_types=(
            [pltpu.VMEM(((npw + 1) * c,), dtype)]
            + [pltpu.VMEM((eb, c), dtype) for _ in range(4)]
            + [pltpu.VMEM((eb,), jnp.int32) for _ in range(4)]
            + [pltpu.VMEM((_CAP,), jnp.int32) for _ in range(2)]
            + [pltpu.SemaphoreType.DMA for _ in range(4)]
        ),
        compiler_params=pltpu.CompilerParams(needs_layout_passes=False),
    )
    def seg(t_hbm, bsrc_hbm, bdstl_hbm, o_hbm,
            own_v, r0, r1, r2, r3, i0, i1, i2, i3,
            bsrc_v, bdstl_v, s0, s1, s2, s3):
        wid = lax.axis_index("s") * _NC + lax.axis_index("c")
        lo = wid * npw
        ident = jnp.full((16,), identity, dtype)

        def memset(q, _):
            own_v[pl.ds(q * 16, 16)] = ident
            return 0

        lax.fori_loop(0, ((npw + 1) * c) // 16, memset, 0)
        pltpu.sync_copy(bsrc_hbm.at[wid], bsrc_v)
        pltpu.sync_copy(bdstl_hbm.at[wid], bdstl_v)
        nb = _CAP // eb

        rings = (r0, r1, r2, r3)
        idxs = (i0, i1, i2, i3)
        sems = (s0, s1, s2, s3)

        def start(b, k):
            for u in range(eb // 16):
                idxs[k][pl.ds(u * 16, 16)] = bsrc_v[pl.ds(b * eb + u * 16, 16)]
            pltpu.async_copy(t_hbm.at[idxs[k]], rings[k], sems[k])

        def wait(k):
            pltpu.make_async_copy(t_hbm.at[idxs[k]],
                                  rings[k], sems[k]).wait()

        def process(b, k):
            rows = rings[k]
            for g in range(eb // 16):
                dvec = bdstl_v[pl.ds(b * eb + g * 16, 16)]
                for i in range(16):
                    base = pl.multiple_of(dvec[i] * c, c)

                    def jbody(j, _):
                        off = pl.multiple_of(j * 16, 16)
                        cur = own_v[pl.ds(base + off, 16)]
                        r = rows[g * 16 + i, pl.ds(off, 16)]
                        own_v[pl.ds(base + off, 16)] = combine(cur, r)
                        return 0

                    lax.fori_loop(0, nj, jbody, 0)

        start(0, 0)
        start(1, 1)
        start(2, 2)

        def quad(bq, _):
            for k in range(4):
                b = bq * 4 + k
                wait(k)
                start(b + 3, (k + 3) % 4)
                process(b, k)
            return 0

        lax.fori_loop(0, (nb - 4) // 4, quad, 0)
        for k in range(4):
            b_tail = nb - 4 + k
            wait(k)
            if k == 0:
                start(nb - 1, 3)
            process(b_tail, k)

        pltpu.sync_copy(own_v.at[pl.ds(0, npw * c)],
                        o_hbm.at[pl.ds(pl.multiple_of(lo * c, c), npw * c)])

    return seg


def _hll_count_f(regs_f):
    s = jnp.sum(jnp.exp2(-regs_f), axis=-1, keepdims=True)
    est = _ALPHA * _HLL_M * _HLL_M / s
    zeros = jnp.sum((regs_f == 0.0).astype(jnp.float32), axis=-1, keepdims=True)
    lin = _HLL_M * jnp.log(_HLL_M / jnp.maximum(zeros, 1.0))
    return jnp.where((est <= 2.5 * _HLL_M) & (zeros > 0), lin, est)


def _edge_body(ms1, ms2, md1, md2, hs1, hs2, hd1, hd2, cs, cd, wl, bl,
               sf_ref, stat_ref):
    def jac(a, b):
        return jnp.mean((a[...] == b[...]).astype(jnp.float32), axis=-1,
                        keepdims=True)

    def un(a, b):
        return _hll_count_f(jnp.maximum(a[...], b[...]).astype(jnp.float32))

    f0 = jac(ms1, md1) * un(hs1, hd1)
    i21 = jac(ms2, md1) * un(hs2, hd1)
    i12 = jac(ms1, md2) * un(hs1, hd2)
    i22 = jac(ms2, md2) * un(hs2, hd2)
    f1 = i21 - f0
    f2 = i12 - f0
    f3 = i22 - f0 - f1 - f2
    c1 = cs[...]
    c2 = cd[...]
    f4 = c2[:, 0:1] - f0 - f1
    f5 = c1[:, 0:1] - f0 - f2
    f6 = c2[:, 1:2] - f1 - f3 - f4
    f7 = c1[:, 1:2] - f2 - f3 - f5
    feats = jnp.concatenate([f0, f1, f2, f3, f4, f5, f6, f7], axis=1)
    feats = jnp.maximum(feats, 0.0)
    sf = jnp.dot(feats, wl[...], preferred_element_type=jnp.float32) + bl[...]
    sf_ref[...] = sf

    @pl.when(pl.program_id(0) == 0)
    def _():
        stat_ref[...] = jnp.zeros_like(stat_ref)

    stat_ref[...] += jnp.concatenate(
        [jnp.sum(sf, axis=0, keepdims=True),
         jnp.sum(sf * sf, axis=0, keepdims=True)], axis=0)


def _edge_stage(ms1, ms2, md1, md2, hs1, hs2, hd1, hd2, cs, cd, Wl, bl):
    e = ms1.shape[0]
    blk = 800 if e % 800 == 0 else e
    grid = e // blk
    mh_spec = pl.BlockSpec((blk, 128), lambda i: (i, 0))
    hl_spec = pl.BlockSpec((blk, 256), lambda i: (i, 0))
    c_spec = pl.BlockSpec((blk, 2), lambda i: (i, 0))
    w_spec = pl.BlockSpec((8, 8), lambda i: (0, 0))
    b_spec = pl.BlockSpec((1, 8), lambda i: (0, 0))
    sf, stat = pl.pallas_call(
        _edge_body,
        grid=(grid,),
        in_specs=[mh_spec] * 4 + [hl_spec] * 4 + [c_spec] * 2 + [w_spec, b_spec],
        out_specs=[pl.BlockSpec((blk, 8), lambda i: (i, 0)),
                   pl.BlockSpec((2, 8), lambda i: (0, 0))],
        out_shape=[jax.ShapeDtypeStruct((e, 8), jnp.float32),
                   jax.ShapeDtypeStruct((2, 8), jnp.float32)],
    )(ms1, ms2, md1, md2, hs1, hs2, hd1, hd2, cs, cd, Wl,
      bl.reshape(1, 8))
    return sf, stat


def kernel(x, edge_index, embed, W1, b1, W2, b2, Wl, bl, gamma, beta, Wf, bf,
           init_minhash, init_hll):
    n = x.shape[0]
    e = edge_index.shape[1]
    src = edge_index[0]
    dst = edge_index[1]
    npw = (n + _NW - 1) // _NW
    npad = npw * _NW

    part = _make_partition(e, npw)
    bsrc, bdstl, bcnt = part(src, dst)

    seg_mh = _make_segreduce(npad, 128, npw, "min", jnp.int32)
    seg_hl = _make_segreduce(npad, 256, npw, "max", jnp.int32)
    seg_add = _make_segreduce(npad, 128, npw, "add", jnp.float32)

    def padn(a):
        return jnp.concatenate(
            [a, jnp.zeros((npad - n,) + a.shape[1:], a.dtype)], axis=0)

    mh0 = padn(init_minhash)
    hl0 = padn(init_hll)
    def run_seg(segk, t, c):
        return segk(t, bsrc, bdstl).reshape(npad, c)

    mh1 = run_seg(seg_mh, mh0, 128)
    mh2 = run_seg(seg_mh, mh1, 128)
    hl1 = run_seg(seg_hl, hl0, 256)
    hl2 = run_seg(seg_hl, hl1, 256)

    cards = jnp.concatenate(
        [_hll_count_f(hl1[:n].astype(jnp.float32)),
         _hll_count_f(hl2[:n].astype(jnp.float32))], axis=1)

    deg = 1.0 + jax.ops.segment_sum(jnp.ones((e,), jnp.float32), dst,
                                    num_segments=n)
    dinv = 1.0 / jnp.sqrt(deg)
    dinv_p = padn(dinv[:, None])

    hw = embed @ W1
    hws = padn(hw) * dinv_p
    s1 = run_seg(seg_add, hws, 128)
    h = jax.nn.relu(s1[:n] * dinv[:, None] + b1)
    hw = h @ W2
    hws = padn(hw) * dinv_p
    s2 = run_seg(seg_add, hws, 128)
    h = jax.nn.relu(s2[:n] * dinv[:, None] + b2)

    sf, stat = _edge_stage(mh1[src], mh2[src], mh1[dst], mh2[dst],
                           hl1[src], hl2[src], hl1[dst], hl2[dst],
                           cards[src], cards[dst], Wl, bl)
    mean = stat[0] / e
    var = stat[1] / e - mean * mean
    sf = gamma * (sf - mean) / jnp.sqrt(var + 1e-5) + beta
    sf = jax.nn.relu(sf)
    sf_nodes = jax.ops.segment_sum(sf, dst, num_segments=n)

    out = jnp.concatenate([h, sf_nodes], axis=1) @ Wf + bf
    return jax.nn.sigmoid(out)
